# baseline (speedup 1.0000x reference)
import functools

import jax
import jax.numpy as jnp
from jax import lax
from jax.experimental import pallas as pl
from jax.experimental.pallas import tpu as pltpu
from jax.experimental.pallas import tpu_sc as plsc

_VALID_DIST = 16
_HID = 128
_B = 32
_N = 100
_ROWS = _B * _N * _N
_NTAB = 2 * 2 * (2 * _VALID_DIST + 3)
_NW = 32
_RPT = _ROWS // _NW
_C = 400
_NCHUNK = _RPT // _C


def _prep_kernel(sp_ref, tn_ref, st_ref, tt_ref, pt_ref, ctab_ref, idx_ref):
    k32 = lax.broadcasted_iota(jnp.int32, (_NTAB, _HID // 4), 0)
    sp_part = jnp.where(k32 // 70 == 0, st_ref[0:1, :], st_ref[1:2, :])
    tn_part = jnp.where((k32 // 35) % 2 == 0, tt_ref[0:1, :], tt_ref[1:2, :])
    row_d = lax.broadcasted_iota(jnp.int32, (_NTAB, 35), 0) % 35
    col_d = lax.broadcasted_iota(jnp.int32, (_NTAB, 35), 1)
    onehot = (row_d == col_d).astype(jnp.float32)
    pos_part = jnp.dot(onehot, pt_ref[...], preferred_element_type=jnp.float32,
                       precision=lax.Precision.HIGHEST)
    ctab_ref[...] = jnp.concatenate([sp_part, tn_part, pos_part], axis=1)

    i = lax.broadcasted_iota(jnp.int32, (_B, _N, _N), 1)
    j = lax.broadcasted_iota(jnp.int32, (_B, _N, _N), 2)
    d = jnp.clip(j - i, -_VALID_DIST - 1, _VALID_DIST + 1) + _VALID_DIST + 1
    idx_ref[...] = sp_ref[...] * 70 + tn_ref[...] * 35 + d


_prep = pl.pallas_call(
    _prep_kernel,
    out_shape=[
        jax.ShapeDtypeStruct((_NTAB, _HID), jnp.float32),
        jax.ShapeDtypeStruct((_B, _N, _N), jnp.int32),
    ],
)


_SL = _C // _N
_NP = 104


@functools.cache
def _make_sc_gather():
    @functools.partial(
        pl.kernel,
        mesh=plsc.VectorSubcoreMesh(core_axis_name="c", subcore_axis_name="s"),
        compiler_params=pltpu.CompilerParams(needs_layout_passes=False),
        out_type=jax.ShapeDtypeStruct((_B, _N, _N, _HID), jnp.float32),
        scratch_types=[
            pltpu.VMEM((_NTAB * _HID,), jnp.float32),
            *[pltpu.VMEM((_C,), jnp.int32) for _ in range(2)],
            *[pltpu.VMEM((_SL, _NP, _HID), jnp.float32) for _ in range(2)],
            *[pltpu.SemaphoreType.DMA for _ in range(4)],
        ],
    )
    def _sc_gather(idx_hbm, ctab_hbm, out_hbm,
                   ctab_v, idx0, idx1, rows0, rows1,
                   ssem0, ssem1, isem0, isem1):
        idxb = (idx0, idx1)
        rowsb = (rows0, rows1)
        ssem = (ssem0, ssem1)
        isem = (isem0, isem1)
        wid = lax.axis_index("s") * 2 + lax.axis_index("c")
        base = wid * _RPT

        pltpu.sync_copy(ctab_hbm, ctab_v)

        def fire_idx(c, pb):
            pltpu.async_copy(
                idx_hbm.at[pl.ds(base + c * _C, _C)], idxb[pb], isem[pb])

        def wait_idx(c, pb):
            pltpu.make_async_copy(
                idx_hbm.at[pl.ds(base + c * _C, _C)], idxb[pb],
                isem[pb]).wait()

        def compute_chunk(c, pb):
            def rg_body(rg, carry):
                off = pl.multiple_of(rg * 16, 16)
                gbv = idxb[pb][pl.ds(off, 16)] * _HID
                gbs = [pl.multiple_of(gbv[l], 16) for l in range(16)]

                def loads(l):
                    return [ctab_v[pl.ds(gbs[l] + k * 16, 16)]
                            for k in range(_HID // 16)]

                def stores(l, vals):
                    r = off + l
                    sl = r // _N
                    rr = r - sl * _N
                    for k in range(_HID // 16):
                        rowsb[pb][sl, rr, pl.ds(k * 16, 16)] = vals[k]

                def store_one(l, k, val):
                    r = off + l
                    sl = r // _N
                    rr = r - sl * _N
                    rowsb[pb][sl, rr, pl.ds(k * 16, 16)] = val

                prev = loads(0)
                for l in range(1, 16):
                    cur = []
                    for k in range(_HID // 16):
                        cur.append(ctab_v[pl.ds(gbs[l] + k * 16, 16)])
                        store_one(l - 1, k, prev[k])
                    prev = cur
                stores(15, prev)
                return carry

            lax.fori_loop(0, _C // 16, rg_body, 0, unroll=False)

        def fire_scatter(c, pb):
            pltpu.async_copy(
                rowsb[pb].at[:, pl.ds(0, _N)],
                out_hbm.at[wid, pl.ds(c * _SL, _SL)], ssem[pb])

        def wait_scatter(pb):
            pltpu.make_async_copy(
                rowsb[pb].at[:, pl.ds(0, _N)],
                out_hbm.at[wid, pl.ds(0, _SL)], ssem[pb]).wait()

        fire_idx(0, 0)
        fire_idx(1, 1)
        wait_idx(0, 0)
        compute_chunk(0, 0)
        fire_scatter(0, 0)
        fire_idx(2, 0)
        wait_idx(1, 1)
        compute_chunk(1, 1)
        fire_scatter(1, 1)

        def body(pi, carry):
            c = 2 + 2 * pi
            fire_idx(c + 1, 1)
            wait_scatter(0)
            wait_idx(c, 0)
            compute_chunk(c, 0)
            fire_scatter(c, 0)
            fire_idx(c + 2, 0)
            wait_scatter(1)
            wait_idx(c + 1, 1)
            compute_chunk(c + 1, 1)
            fire_scatter(c + 1, 1)
            return carry

        lax.fori_loop(0, (_NCHUNK - 2) // 2, body, 0, unroll=False)
        wait_scatter(0)
        wait_idx(_NCHUNK - 1, 0)
        compute_chunk(_NCHUNK - 1, 0)
        fire_scatter(_NCHUNK - 1, 0)
        wait_scatter(0)
        wait_scatter(1)

    return _sc_gather


def kernel(speaker, turn, speaker_table, turn_table, position_table):
    ctab, idx = _prep(
        speaker.astype(jnp.int32), turn.astype(jnp.int32),
        speaker_table, turn_table, position_table,
    )
    return _make_sc_gather()(idx.reshape(_ROWS), ctab.reshape(_NTAB * _HID))
